# Initial kernel scaffold; baseline (speedup 1.0000x reference)
import jax
import jax.numpy as jnp
from jax.experimental import pallas as pl


def kernel(x, edge_index, idx, W, att_src, att_dst, bias):
    raise NotImplementedError("write your pallas kernel here")


# speedup vs baseline: 21.9995x; 21.9995x over previous
import functools

import jax
import jax.numpy as jnp
from jax import lax
from jax.experimental import pallas as pl
from jax.experimental.pallas import tpu as pltpu
from jax.experimental.pallas import tpu_sc as plsc

N_NODES = 10000
N_PAD = 10240
D = 128
C = 128
E = 320000
NC, NS = 2, 16
NW = NC * NS
CHUNK = 128
CHUNKS_PER_TILE = 79
E_PAD = CHUNKS_PER_TILE * NW * CHUNK
ROWS_PER_TILE = N_PAD // NS
BLK = 1024


def _prep_body(x_ref, wt_ref, vs_ref, vd_ref, xs_ref, asrc_ref, adst_ref):
    xs = jnp.dot(x_ref[...], wt_ref[...], preferred_element_type=jnp.float32)
    xs_ref[...] = xs
    asrc_ref[...] = jnp.sum(xs * vs_ref[...][None, :], axis=1)
    adst_ref[...] = jnp.sum(xs * vd_ref[...][None, :], axis=1)


def _prep(x_pad, wt, vs, vd):
    grid = (N_PAD // BLK,)
    return pl.pallas_call(
        _prep_body,
        grid=grid,
        in_specs=[
            pl.BlockSpec((BLK, D), lambda i: (i, 0)),
            pl.BlockSpec((D, C), lambda i: (0, 0)),
            pl.BlockSpec((C,), lambda i: (0,)),
            pl.BlockSpec((C,), lambda i: (0,)),
        ],
        out_specs=[
            pl.BlockSpec((BLK, C), lambda i: (i, 0)),
            pl.BlockSpec((BLK,), lambda i: (i,)),
            pl.BlockSpec((BLK,), lambda i: (i,)),
        ],
        out_shape=[
            jax.ShapeDtypeStruct((N_PAD, C), jnp.float32),
            jax.ShapeDtypeStruct((N_PAD,), jnp.float32),
            jax.ShapeDtypeStruct((N_PAD,), jnp.float32),
        ],
    )(x_pad, wt, vs, vd)


def _sc_body(xs_hbm, asrc_hbm, adst_hbm, src_hbm, dst_hbm,
             acc_out, den_out,
             asrc_v, adst_v, src_v, dst_v, e_v, rows_v,
             acc_sh, den_sh):
    cid = lax.axis_index("c")
    sid = lax.axis_index("s")
    wid = cid * NS + sid

    pltpu.sync_copy(asrc_hbm, asrc_v)
    pltpu.sync_copy(adst_hbm, adst_v)

    zeros16 = jnp.zeros((16,), jnp.float32)

    def _zrow(i, carry):
        for q in range(C // 16):
            rows_v[i, pl.ds(q * 16, 16)] = zeros16
        return carry

    lax.fori_loop(0, CHUNK, _zrow, 0)
    for q in range(CHUNK // 16):
        e_v[pl.ds(q * 16, 16)] = zeros16
    for t in range(ROWS_PER_TILE // CHUNK):
        r0 = sid * ROWS_PER_TILE + t * CHUNK
        pltpu.sync_copy(rows_v, acc_sh.at[pl.ds(r0, CHUNK)])
        pltpu.sync_copy(e_v, den_sh.at[pl.ds(r0, CHUNK)])
    plsc.subcore_barrier()

    def _chunk(j, carry):
        base = (wid + NW * j) * CHUNK
        pltpu.sync_copy(src_hbm.at[pl.ds(base, CHUNK)], src_v)
        pltpu.sync_copy(dst_hbm.at[pl.ds(base, CHUNK)], dst_v)

        for g in range(CHUNK // 16):
            si = src_v[pl.ds(g * 16, 16)]
            di = dst_v[pl.ds(g * 16, 16)]
            a1 = plsc.load_gather(asrc_v, [si])
            a2 = plsc.load_gather(adst_v, [di])
            s = a1 + a2
            s = jnp.where(s >= 0.0, s, 0.2 * s)
            e = jnp.where(si != di, jnp.exp(s), 0.0)
            e_v[pl.ds(g * 16, 16)] = e

        pltpu.sync_copy(xs_hbm.at[src_v], rows_v)

        def _scale(i, carry2):
            spl = plsc.load_gather(e_v, [jnp.full((16,), 0, jnp.int32) + i])
            for q in range(C // 16):
                sl = pl.ds(q * 16, 16)
                rows_v[i, sl] = rows_v[i, sl] * spl
            return carry2

        lax.fori_loop(0, CHUNK, _scale, 0)

        pltpu.sync_copy(rows_v, acc_sh.at[dst_v], add=True)
        pltpu.sync_copy(e_v, den_sh.at[dst_v], add=True)
        return carry

    lax.fori_loop(0, CHUNKS_PER_TILE, _chunk, 0)
    plsc.subcore_barrier()

    for t in range(ROWS_PER_TILE // CHUNK):
        r0 = sid * ROWS_PER_TILE + t * CHUNK
        pltpu.sync_copy(acc_sh.at[pl.ds(r0, CHUNK)], rows_v)
        pltpu.sync_copy(rows_v, acc_out.at[cid, pl.ds(r0, CHUNK)])
        pltpu.sync_copy(den_sh.at[pl.ds(r0, CHUNK)], e_v)
        pltpu.sync_copy(e_v, den_out.at[cid, pl.ds(r0, CHUNK)])


_sc_edges = functools.partial(
    pl.kernel,
    out_type=[
        jax.ShapeDtypeStruct((NC, N_PAD, C), jnp.float32),
        jax.ShapeDtypeStruct((NC, N_PAD), jnp.float32),
    ],
    mesh=plsc.VectorSubcoreMesh(core_axis_name="c", subcore_axis_name="s"),
    compiler_params=pltpu.CompilerParams(
        needs_layout_passes=False, use_tc_tiling_on_sc=False,
        internal_scratch_in_bytes=128 * 1024),
    scratch_types=[
        pltpu.VMEM((N_PAD,), jnp.float32),
        pltpu.VMEM((N_PAD,), jnp.float32),
        pltpu.VMEM((CHUNK,), jnp.int32),
        pltpu.VMEM((CHUNK,), jnp.int32),
        pltpu.VMEM((CHUNK,), jnp.float32),
        pltpu.VMEM((CHUNK, C), jnp.float32),
        pltpu.VMEM_SHARED((N_PAD, C), jnp.float32),
        pltpu.VMEM_SHARED((N_PAD,), jnp.float32),
    ],
)(_sc_body)


def _combine_body(acc0_ref, acc1_ref, den0_ref, den1_ref, asrc_ref, adst_ref,
                  xs_ref, bias_ref, out_ref):
    s = asrc_ref[...] + adst_ref[...]
    s = jnp.where(s >= 0.0, s, 0.2 * s)
    es = jnp.exp(s)
    den = den0_ref[...] + den1_ref[...] + es
    num = acc0_ref[...] + acc1_ref[...] + es[:, None] * xs_ref[...]
    out_ref[...] = num / (den[:, None] + 1e-16) + bias_ref[...][None, :]


def _combine(acc0, acc1, den0, den1, asrc, adst, xs, bias):
    grid = (N_PAD // BLK,)
    mat = pl.BlockSpec((BLK, C), lambda i: (i, 0))
    vec = pl.BlockSpec((BLK,), lambda i: (i,))
    return pl.pallas_call(
        _combine_body,
        grid=grid,
        in_specs=[mat, mat, vec, vec, vec, vec, mat,
                  pl.BlockSpec((C,), lambda i: (0,))],
        out_specs=mat,
        out_shape=jax.ShapeDtypeStruct((N_PAD, C), jnp.float32),
    )(acc0, acc1, den0, den1, asrc, adst, xs, bias)


def kernel(x, edge_index, idx, W, att_src, att_dst, bias):
    sign = jnp.where(idx == 1, jnp.float32(-1.0), jnp.float32(1.0))
    vs = (sign * att_src).reshape(C).astype(jnp.float32)
    vd = (sign * att_dst).reshape(C).astype(jnp.float32)
    x_pad = jnp.concatenate(
        [x, jnp.zeros((N_PAD - N_NODES, D), jnp.float32)], axis=0)
    src = jnp.concatenate(
        [edge_index[0], jnp.zeros((E_PAD - E,), edge_index.dtype)])
    dst = jnp.concatenate(
        [edge_index[1], jnp.zeros((E_PAD - E,), edge_index.dtype)])

    xs, asrc, adst = _prep(x_pad, W.T, vs, vd)
    acc, den = _sc_edges(xs, asrc, adst, src, dst)
    out = _combine(acc[0], acc[1], den[0], den[1], asrc, adst, xs, bias)
    return out[:N_NODES]
